# baseline (speedup 1.0000x reference)
import functools

import jax
import jax.numpy as jnp
from jax import lax
from jax.experimental import pallas as pl
from jax.experimental.pallas import tpu as pltpu
from jax.experimental.pallas import tpu_sc as plsc

_B = 1024
_D = 1024
_N_POP = 256
_T = 256
_TAU = 10.0
_SCALE = _T * _TAU / (_TAU + 1.0)

_NC = 2
_NS = 16
_NW = _NC * _NS
_HALF_WORDS = (_T // 2) * _N_POP
_PAD = 16
_ITEMS = 2 * _B
_IPW = _ITEMS // _NW


def _offsets_body(x_ref, w_ref, b_ref, off_ref):
    z = jnp.dot(x_ref[...], w_ref[...], preferred_element_type=jnp.float32)
    intensity = jax.nn.sigmoid(z + b_ref[...])
    st = jnp.clip(((1.0 - intensity) * _SCALE).astype(jnp.int32), 0, _T - 1)
    n_iota = lax.broadcasted_iota(jnp.int32, (_B, _N_POP), 1)
    pad = _HALF_WORDS + (n_iota & 15)
    in0 = st < (_T // 2)
    off0 = jnp.where(in0, st * _N_POP + n_iota, pad)
    off1 = jnp.where(in0, pad, (st - (_T // 2)) * _N_POP + n_iota)
    off_ref[...] = jnp.stack([off0, off1], axis=1)


def _sc_body(off_ref, out_ref, blk_a, blk_b, idx_v, sem_a, sem_b):
    c = lax.axis_index("c")
    s = lax.axis_index("s")
    base = (s * _NC + c) * _IPW

    zeros16 = jnp.zeros((16,), jnp.float32)
    ones16 = jnp.ones((16,), jnp.float32)

    pltpu.sync_copy(off_ref.at[pl.ds(base * _N_POP, _IPW * _N_POP)], idx_v)

    def memset(blk):
        def body(i, carry):
            blk[pl.ds(i * 16, 16)] = zeros16
            return carry
        lax.fori_loop(0, (_HALF_WORDS + _PAD) // 16, body, 0)

    memset(blk_a)
    memset(blk_b)

    def scatter(blk, kk, val16):
        for j in range(_N_POP // 16):
            ix = idx_v[pl.ds(kk * _N_POP + j * 16, 16)]
            plsc.store_scatter(blk, [ix], val16)

    def fire(blk, kk, sem):
        return pltpu.async_copy(
            blk.at[pl.ds(0, _HALF_WORDS)], out_ref.at[base + kk], sem)

    scatter(blk_a, 0, ones16)
    fire(blk_a, 0, sem_a)
    scatter(blk_b, 1, ones16)
    fire(blk_b, 1, sem_b)

    def loop(i, carry):
        ka, kb = 2 * i, 2 * i + 1
        pltpu.make_async_copy(
            blk_a.at[pl.ds(0, _HALF_WORDS)], out_ref.at[base + ka - 2], sem_a
        ).wait()
        scatter(blk_a, ka - 2, zeros16)
        scatter(blk_a, ka, ones16)
        fire(blk_a, ka, sem_a)
        pltpu.make_async_copy(
            blk_b.at[pl.ds(0, _HALF_WORDS)], out_ref.at[base + kb - 2], sem_b
        ).wait()
        scatter(blk_b, kb - 2, zeros16)
        scatter(blk_b, kb, ones16)
        fire(blk_b, kb, sem_b)
        return carry

    lax.fori_loop(1, _IPW // 2, loop, 0)

    pltpu.make_async_copy(
        blk_a.at[pl.ds(0, _HALF_WORDS)], out_ref.at[base + _IPW - 2], sem_a
    ).wait()
    pltpu.make_async_copy(
        blk_b.at[pl.ds(0, _HALF_WORDS)], out_ref.at[base + _IPW - 1], sem_b
    ).wait()


@functools.partial(jax.jit)
def kernel(x, W, b):
    wt = W.T
    b2 = b.reshape(1, _N_POP)
    offs = pl.pallas_call(
        _offsets_body,
        grid=(1,),
        in_specs=[
            pl.BlockSpec((_B, _D), lambda i: (0, 0)),
            pl.BlockSpec((_D, _N_POP), lambda i: (0, 0)),
            pl.BlockSpec((1, _N_POP), lambda i: (0, 0)),
        ],
        out_specs=pl.BlockSpec((_B, 2, _N_POP), lambda i: (0, 0, 0)),
        out_shape=jax.ShapeDtypeStruct((_B, 2, _N_POP), jnp.int32),
    )(x, wt, b2)

    sc = pl.kernel(
        _sc_body,
        out_type=jax.ShapeDtypeStruct((_ITEMS, _HALF_WORDS), jnp.float32),
        mesh=plsc.VectorSubcoreMesh(core_axis_name="c", subcore_axis_name="s"),
        scratch_types=[
            pltpu.VMEM((_HALF_WORDS + _PAD,), jnp.float32),
            pltpu.VMEM((_HALF_WORDS + _PAD,), jnp.float32),
            pltpu.VMEM((_IPW * _N_POP,), jnp.int32),
            pltpu.SemaphoreType.DMA,
            pltpu.SemaphoreType.DMA,
        ],
        compiler_params=pltpu.CompilerParams(needs_layout_passes=False),
    )
    flat = sc(offs.reshape(_ITEMS * _N_POP))
    return flat.reshape(_B, _T, _N_POP)


# speedup vs baseline: 5.0026x; 5.0026x over previous
import functools

import jax
import jax.numpy as jnp
from jax.experimental import pallas as pl

_B = 1024
_D = 1024
_N_POP = 256
_T = 256
_TAU = 10.0
_SCALE = _T * _TAU / (_TAU + 1.0)

_BB = 32


def _onehot_block(x_ref, w_ref, b_ref, out_ref):
    z = jnp.dot(x_ref[...], w_ref[...], preferred_element_type=jnp.float32)
    intensity = jax.nn.sigmoid(z + b_ref[...])
    st = jnp.clip(((1.0 - intensity) * _SCALE).astype(jnp.int32), 0, _T - 1)
    t_iota = jax.lax.broadcasted_iota(jnp.int32, (_BB, _T, _N_POP), 1)
    out_ref[...] = (t_iota == st[:, None, :]).astype(jnp.float32)


@functools.partial(jax.jit)
def kernel(x, W, b):
    wt = W.T
    b2 = b.reshape(1, _N_POP)
    return pl.pallas_call(
        _onehot_block,
        grid=(_B // _BB,),
        in_specs=[
            pl.BlockSpec((_BB, _D), lambda i: (i, 0)),
            pl.BlockSpec((_D, _N_POP), lambda i: (0, 0)),
            pl.BlockSpec((1, _N_POP), lambda i: (0, 0)),
        ],
        out_specs=pl.BlockSpec((_BB, _T, _N_POP), lambda i: (i, 0, 0)),
        out_shape=jax.ShapeDtypeStruct((_B, _T, _N_POP), jnp.float32),
    )(x, wt, b2)
